# baseline (speedup 1.0000x reference)
import functools

import jax
import jax.numpy as jnp
from jax import lax
from jax.experimental import pallas as pl
from jax.experimental.pallas import tpu as pltpu
from jax.experimental.pallas import tpu_sc as plsc

EPSILON = 5.0
K_SEL = 10
NEG_INF = jnp.float32(-3.0e38)
BIG_I32 = jnp.int32(1 << 30)

_BM = 512


def _ka_body(x_ref, w1_ref, b1_ref, w2_ref, h_ref, p_ref):
    x = x_ref[...]
    h = jax.lax.dot_general(
        x, w1_ref[...], (((1,), (0,)), ((), ())),
        preferred_element_type=jnp.float32)
    h = jnp.maximum(h + b1_ref[...], 0.0)
    h_ref[...] = h
    p_ref[...] = jax.lax.dot_general(
        h, w2_ref[...], (((1,), (0,)), ((), ())),
        preferred_element_type=jnp.float32)


def _kb_body(hi_ref, hj_ref, dist_ref, sums_ref):
    j = pl.program_id(1)
    hi = hi_ref[...]
    hj = hj_ref[...]
    aa = jnp.sum(hi * hi, axis=1, keepdims=True)
    bb = jnp.sum(hj * hj, axis=1)
    g = jax.lax.dot_general(
        hi, hj, (((1,), (1,)), ((), ())),
        preferred_element_type=jnp.float32)
    d2 = (aa + bb[None, :]) - 2.0 * g
    dist = jnp.sqrt(jnp.maximum(d2, 0.0))
    dist_ref[...] = dist
    s1 = jnp.sum(dist)
    s2 = jnp.sum(dist * dist)
    lanes = lax.broadcasted_iota(jnp.int32, (1, 1, 128), 2)
    v = jnp.where(lanes == 0, s1, jnp.where(lanes == 1, s2, 0.0))

    @pl.when(j == 0)
    def _():
        sums_ref[...] = v

    @pl.when(j != 0)
    def _():
        sums_ref[...] += v


def _kc_body(scal_ref, dist_ref, score_ref, cnt_ref):
    j = pl.program_id(1)
    nj = pl.num_programs(1)
    d = dist_ref[...]
    mask = d < EPSILON

    @pl.when(j == 0)
    def _():
        cnt_ref[...] = jnp.zeros_like(cnt_ref)
        score_ref[...] = jnp.zeros_like(score_ref)

    cnt_ref[...] += jnp.sum(mask.astype(jnp.float32), axis=1, keepdims=True)

    @pl.when(jnp.any(mask))
    def _():
        gk = jnp.exp(-(d * d) * scal_ref[0])
        score_ref[...] += jnp.sum(jnp.where(mask, gk, 0.0), axis=1,
                                  keepdims=True)

    @pl.when(j == nj - 1)
    def _():
        cnt = cnt_ref[...]
        typ = jnp.where(cnt > 0.0, score_ref[...] / jnp.maximum(cnt, 1.0), 0.0)
        score_ref[...] = scal_ref[1] * typ


def _features(instances, W1, b1r, W2):
    n, d_in = instances.shape
    d_h = W1.shape[1]
    n_cls = W2.shape[1]
    nb = n // _BM
    return pl.pallas_call(
        _ka_body,
        grid=(nb,),
        in_specs=[
            pl.BlockSpec((_BM, d_in), lambda i: (i, 0)),
            pl.BlockSpec((d_in, d_h), lambda i: (0, 0)),
            pl.BlockSpec((1, d_h), lambda i: (0, 0)),
            pl.BlockSpec((d_h, n_cls), lambda i: (0, 0)),
        ],
        out_specs=[
            pl.BlockSpec((_BM, d_h), lambda i: (i, 0)),
            pl.BlockSpec((_BM, n_cls), lambda i: (i, 0)),
        ],
        out_shape=[
            jax.ShapeDtypeStruct((n, d_h), jnp.float32),
            jax.ShapeDtypeStruct((n, n_cls), jnp.float32),
        ],
    )(instances, W1, b1r, W2)


def _distances(h):
    n, d_h = h.shape
    nb = n // _BM
    return pl.pallas_call(
        _kb_body,
        grid=(nb, nb),
        in_specs=[
            pl.BlockSpec((_BM, d_h), lambda i, j: (i, 0)),
            pl.BlockSpec((_BM, d_h), lambda i, j: (j, 0)),
        ],
        out_specs=[
            pl.BlockSpec((_BM, _BM), lambda i, j: (i, j)),
            pl.BlockSpec((1, 1, 128), lambda i, j: (i, 0, 0)),
        ],
        out_shape=[
            jax.ShapeDtypeStruct((n, n), jnp.float32),
            jax.ShapeDtypeStruct((nb, 1, 128), jnp.float32),
        ],
    )(h, h)


def _typicality(scal, dist):
    n = dist.shape[0]
    nb = n // _BM
    return pl.pallas_call(
        _kc_body,
        grid=(nb, nb),
        in_specs=[
            pl.BlockSpec(memory_space=pltpu.SMEM),
            pl.BlockSpec((_BM, _BM), lambda i, j: (i, j)),
        ],
        out_specs=pl.BlockSpec((_BM, 1), lambda i, j: (i, 0)),
        out_shape=jax.ShapeDtypeStruct((n, 1), jnp.float32),
        scratch_shapes=[pltpu.VMEM((_BM, 1), jnp.float32)],
    )(scal, dist)


_NC, _NS = 1, 16
_NW = _NC * _NS


def _shuffle(x, idx):
    return x.at[idx].get(mode="promise_in_bounds")


def _all_reduce(v, op, lane):
    for s in (1, 2, 4, 8):
        v = op(v, _shuffle(v, lane ^ s))
    return v


def _sc_select(score, p_flat, b2p, n_cls):
    n = score.shape[0]
    chunk = n // _NW
    nregs = chunk // 16
    mesh = plsc.VectorSubcoreMesh(core_axis_name="c", subcore_axis_name="s",
                                  num_cores=_NC)

    @functools.partial(
        pl.kernel,
        mesh=mesh,
        compiler_params=pltpu.CompilerParams(needs_layout_passes=False),
        out_type=jax.ShapeDtypeStruct((16,), jnp.float32),
        scratch_types=[
            pltpu.VMEM((chunk,), jnp.float32),
            pltpu.VMEM((16,), jnp.float32),
            pltpu.VMEM((16,), jnp.int32),
            pltpu.VMEM_SHARED((_NW * 16,), jnp.float32),
            pltpu.VMEM_SHARED((_NW * 16,), jnp.int32),
            pltpu.VMEM((_NW * 16,), jnp.float32),
            pltpu.VMEM((_NW * 16,), jnp.int32),
            pltpu.VMEM((n * n_cls,), jnp.float32),
            pltpu.VMEM((16,), jnp.float32),
        ],
    )
    def sc_kernel(score_hbm, p_hbm, b2_hbm, out_hbm,
                  score_v, cv_v, ci_v, shv, shi, av, ai, p_v, st_v):
        wid = lax.axis_index("s")
        lane = lax.iota(jnp.int32, 16)
        pltpu.sync_copy(score_hbm.at[pl.ds(wid * chunk, chunk)], score_v)

        cv = jnp.full((16,), NEG_INF, jnp.float32)
        ci = jnp.full((16,), BIG_I32, jnp.int32)
        for t in range(K_SEL):
            best_v = jnp.full((16,), NEG_INF, jnp.float32)
            best_i = jnp.full((16,), BIG_I32, jnp.int32)
            for r in range(nregs):
                v = score_v[pl.ds(r * 16, 16)]
                idx = lane + (r * 16)
                upd = (v > best_v) | ((v == best_v) & (idx < best_i))
                best_v = jnp.where(upd, v, best_v)
                best_i = jnp.where(upd, idx, best_i)
            m = _all_reduce(best_v, jnp.maximum, lane)
            sel = _all_reduce(
                jnp.where(best_v == m, best_i, BIG_I32), jnp.minimum, lane)
            cv = jnp.where(lane == t, m, cv)
            ci = jnp.where(lane == t, sel + wid * chunk, ci)
            plsc.store_scatter(
                score_v, [sel],
                jnp.full((16,), NEG_INF, jnp.float32), mask=lane == 0)
        cv_v[...] = cv
        ci_v[...] = ci
        pltpu.sync_copy(cv_v, shv.at[pl.ds(wid * 16, 16)])
        pltpu.sync_copy(ci_v, shi.at[pl.ds(wid * 16, 16)])
        plsc.subcore_barrier()

        @pl.when(wid == 0)
        def _():
            pltpu.sync_copy(shv, av)
            pltpu.sync_copy(shi, ai)
            pltpu.sync_copy(p_hbm, p_v)
            selv = jnp.zeros((16,), jnp.int32)
            for t in range(K_SEL):
                best_v = jnp.full((16,), NEG_INF, jnp.float32)
                best_i = jnp.full((16,), BIG_I32, jnp.int32)
                for r in range(_NW):
                    v = av[pl.ds(r * 16, 16)]
                    i = ai[pl.ds(r * 16, 16)]
                    upd = (v > best_v) | ((v == best_v) & (i < best_i))
                    best_v = jnp.where(upd, v, best_v)
                    best_i = jnp.where(upd, i, best_i)
                m = _all_reduce(best_v, jnp.maximum, lane)
                sel = _all_reduce(
                    jnp.where(best_v == m, best_i, BIG_I32), jnp.minimum, lane)
                selv = jnp.where(lane == t, sel, selv)
                for r in range(_NW):
                    i = ai[pl.ds(r * 16, 16)]
                    v = av[pl.ds(r * 16, 16)]
                    av[pl.ds(r * 16, 16)] = jnp.where(i == sel, NEG_INF, v)
            msk = lane < K_SEL
            acc = jnp.zeros((16,), jnp.float32)
            for c in range(n_cls):
                g = plsc.load_gather(p_v, [selv * n_cls + c])
                s = _all_reduce(jnp.where(msk, g, 0.0), jnp.add, lane)
                acc = jnp.where(lane == c, s / jnp.float32(K_SEL), acc)
            pltpu.sync_copy(b2_hbm, st_v)
            st_v[...] = st_v[...] + acc
            pltpu.sync_copy(st_v, out_hbm)

    return sc_kernel(score, p_flat, b2p)


def kernel(instances, bag_label, W1, b1, W2, b2):
    n = instances.shape[0]
    h, p = _features(instances, W1, b1.reshape(1, -1), W2)
    dist, sums = _distances(h)

    tot = jnp.sum(sums, axis=(0, 1))
    m = float(n) * float(n)
    mean = tot[0] / m
    s = jnp.sqrt(jnp.maximum(tot[1] / m - mean * mean, 0.0))
    s = jnp.where(s < 1e-6, mean + 1e-6, s)
    band = jnp.maximum(1.06 * s * (float(n) ** (-0.2)), 0.001)
    inv2b2 = 1.0 / (2.0 * band * band)
    sign = jnp.where(jnp.asarray(bag_label) == 1, 1.0, -1.0)
    scal = jnp.stack([inv2b2, sign]).astype(jnp.float32)

    score = _typicality(scal, dist).reshape(n)
    out16 = _sc_select(score, p.reshape(-1), jnp.pad(b2, (0, 16 - b2.shape[0])),
                       p.shape[1])
    return out16[: b2.shape[0]]


# speedup vs baseline: 1.0054x; 1.0054x over previous
import functools

import jax
import jax.numpy as jnp
from jax import lax
from jax.experimental import pallas as pl
from jax.experimental.pallas import tpu as pltpu
from jax.experimental.pallas import tpu_sc as plsc

EPSILON = 5.0
K_SEL = 10
NEG_INF = jnp.float32(-3.0e38)
BIG_I32 = jnp.int32(1 << 30)

_BM = 512


def _ka_body(x_ref, w1_ref, b1_ref, w2_ref, h_ref, p_ref):
    x = x_ref[...]
    h = jax.lax.dot_general(
        x, w1_ref[...], (((1,), (0,)), ((), ())),
        preferred_element_type=jnp.float32)
    h = jnp.maximum(h + b1_ref[...], 0.0)
    h_ref[...] = h
    p_ref[...] = jax.lax.dot_general(
        h, w2_ref[...], (((1,), (0,)), ((), ())),
        preferred_element_type=jnp.float32)


def _kbc_body(sign_ref, h_ref, score_ref, cnt_v, sc_v, sums_ref, band_ref):
    ph = pl.program_id(0)
    i = pl.program_id(1)
    j = pl.program_id(2)
    nj = pl.num_programs(2)
    n = h_ref.shape[0]

    hi = h_ref[pl.ds(i * _BM, _BM), :]
    hj = h_ref[pl.ds(j * _BM, _BM), :]
    aa = jnp.sum(hi * hi, axis=1, keepdims=True)
    bb = jnp.sum(hj * hj, axis=1)
    g = jax.lax.dot_general(
        hi, hj, (((1,), (1,)), ((), ())),
        preferred_element_type=jnp.float32)
    d2 = (aa + bb[None, :]) - 2.0 * g
    d = jnp.sqrt(jnp.maximum(d2, 0.0))

    @pl.when(ph == 0)
    def _():
        s1 = jnp.sum(d)
        s2 = jnp.sum(d * d)

        @pl.when((i == 0) & (j == 0))
        def _():
            sums_ref[0] = s1
            sums_ref[1] = s2

        @pl.when((i != 0) | (j != 0))
        def _():
            sums_ref[0] += s1
            sums_ref[1] += s2

    @pl.when(ph == 1)
    def _():
        @pl.when((i == 0) & (j == 0))
        def _():
            m = jnp.float32(n) * jnp.float32(n)
            mean = sums_ref[0] / m
            var = sums_ref[1] / m - mean * mean
            s = jnp.sqrt(jnp.maximum(var, 0.0))
            s = jnp.where(s < 1e-6, mean + 1e-6, s)
            band = jnp.maximum(1.06 * s * (float(n) ** (-0.2)), 0.001)
            band_ref[0] = 1.0 / (2.0 * band * band)

        mask = d < EPSILON

        @pl.when(j == 0)
        def _():
            cnt_v[...] = jnp.zeros_like(cnt_v)
            sc_v[...] = jnp.zeros_like(sc_v)

        cnt_v[...] += jnp.sum(mask.astype(jnp.float32), axis=1, keepdims=True)

        @pl.when(jnp.any(mask))
        def _():
            gk = jnp.exp(-(d * d) * band_ref[0])
            sc_v[...] += jnp.sum(jnp.where(mask, gk, 0.0), axis=1,
                                 keepdims=True)

        @pl.when(j == nj - 1)
        def _():
            cnt = cnt_v[...]
            typ = jnp.where(cnt > 0.0, sc_v[...] / jnp.maximum(cnt, 1.0), 0.0)
            score_ref[...] = sign_ref[0] * typ


def _features(instances, W1, b1r, W2):
    n, d_in = instances.shape
    d_h = W1.shape[1]
    n_cls = W2.shape[1]
    nb = n // _BM
    return pl.pallas_call(
        _ka_body,
        grid=(nb,),
        in_specs=[
            pl.BlockSpec((_BM, d_in), lambda i: (i, 0)),
            pl.BlockSpec((d_in, d_h), lambda i: (0, 0)),
            pl.BlockSpec((1, d_h), lambda i: (0, 0)),
            pl.BlockSpec((d_h, n_cls), lambda i: (0, 0)),
        ],
        out_specs=[
            pl.BlockSpec((_BM, d_h), lambda i: (i, 0)),
            pl.BlockSpec((_BM, n_cls), lambda i: (i, 0)),
        ],
        out_shape=[
            jax.ShapeDtypeStruct((n, d_h), jnp.float32),
            jax.ShapeDtypeStruct((n, n_cls), jnp.float32),
        ],
    )(instances, W1, b1r, W2)


def _typicality(sign, h):
    n, d_h = h.shape
    nb = n // _BM
    return pl.pallas_call(
        _kbc_body,
        grid=(2, nb, nb),
        in_specs=[
            pl.BlockSpec(memory_space=pltpu.SMEM),
            pl.BlockSpec((n, d_h), lambda p, i, j: (0, 0)),
        ],
        out_specs=pl.BlockSpec((_BM, 1), lambda p, i, j: (i, 0)),
        out_shape=jax.ShapeDtypeStruct((n, 1), jnp.float32),
        scratch_shapes=[
            pltpu.VMEM((_BM, 1), jnp.float32),
            pltpu.VMEM((_BM, 1), jnp.float32),
            pltpu.SMEM((2,), jnp.float32),
            pltpu.SMEM((1,), jnp.float32),
        ],
    )(sign, h)


_NC, _NS = 1, 16
_NW = _NC * _NS


def _shuffle(x, idx):
    return x.at[idx].get(mode="promise_in_bounds")


def _all_reduce(v, op, lane):
    for s in (1, 2, 4, 8):
        v = op(v, _shuffle(v, lane ^ s))
    return v


def _sc_select(score, p_flat, b2p, n_cls):
    n = score.shape[0]
    chunk = n // _NW
    nregs = chunk // 16
    mesh = plsc.VectorSubcoreMesh(core_axis_name="c", subcore_axis_name="s",
                                  num_cores=_NC)

    @functools.partial(
        pl.kernel,
        mesh=mesh,
        compiler_params=pltpu.CompilerParams(needs_layout_passes=False),
        out_type=jax.ShapeDtypeStruct((16,), jnp.float32),
        scratch_types=[
            pltpu.VMEM((chunk,), jnp.float32),
            pltpu.VMEM((16,), jnp.float32),
            pltpu.VMEM((16,), jnp.int32),
            pltpu.VMEM_SHARED((_NW * 16,), jnp.float32),
            pltpu.VMEM_SHARED((_NW * 16,), jnp.int32),
            pltpu.VMEM((_NW * 16,), jnp.float32),
            pltpu.VMEM((_NW * 16,), jnp.int32),
            pltpu.VMEM((n * n_cls,), jnp.float32),
            pltpu.VMEM((16,), jnp.float32),
        ],
    )
    def sc_kernel(score_hbm, p_hbm, b2_hbm, out_hbm,
                  score_v, cv_v, ci_v, shv, shi, av, ai, p_v, st_v):
        wid = lax.axis_index("s")
        lane = lax.iota(jnp.int32, 16)
        pltpu.sync_copy(score_hbm.at[pl.ds(wid * chunk, chunk)], score_v)

        cv = jnp.full((16,), NEG_INF, jnp.float32)
        ci = jnp.full((16,), BIG_I32, jnp.int32)
        for t in range(K_SEL):
            best_v = jnp.full((16,), NEG_INF, jnp.float32)
            best_i = jnp.full((16,), BIG_I32, jnp.int32)
            for r in range(nregs):
                v = score_v[pl.ds(r * 16, 16)]
                idx = lane + (r * 16)
                upd = (v > best_v) | ((v == best_v) & (idx < best_i))
                best_v = jnp.where(upd, v, best_v)
                best_i = jnp.where(upd, idx, best_i)
            m = _all_reduce(best_v, jnp.maximum, lane)
            sel = _all_reduce(
                jnp.where(best_v == m, best_i, BIG_I32), jnp.minimum, lane)
            cv = jnp.where(lane == t, m, cv)
            ci = jnp.where(lane == t, sel + wid * chunk, ci)
            plsc.store_scatter(
                score_v, [sel],
                jnp.full((16,), NEG_INF, jnp.float32), mask=lane == 0)
        cv_v[...] = cv
        ci_v[...] = ci
        pltpu.sync_copy(cv_v, shv.at[pl.ds(wid * 16, 16)])
        pltpu.sync_copy(ci_v, shi.at[pl.ds(wid * 16, 16)])
        plsc.subcore_barrier()

        @pl.when(wid == 0)
        def _():
            pltpu.sync_copy(shv, av)
            pltpu.sync_copy(shi, ai)
            pltpu.sync_copy(p_hbm, p_v)
            selv = jnp.zeros((16,), jnp.int32)
            for t in range(K_SEL):
                best_v = jnp.full((16,), NEG_INF, jnp.float32)
                best_i = jnp.full((16,), BIG_I32, jnp.int32)
                for r in range(_NW):
                    v = av[pl.ds(r * 16, 16)]
                    i = ai[pl.ds(r * 16, 16)]
                    upd = (v > best_v) | ((v == best_v) & (i < best_i))
                    best_v = jnp.where(upd, v, best_v)
                    best_i = jnp.where(upd, i, best_i)
                m = _all_reduce(best_v, jnp.maximum, lane)
                sel = _all_reduce(
                    jnp.where(best_v == m, best_i, BIG_I32), jnp.minimum, lane)
                selv = jnp.where(lane == t, sel, selv)
                for r in range(_NW):
                    i = ai[pl.ds(r * 16, 16)]
                    v = av[pl.ds(r * 16, 16)]
                    av[pl.ds(r * 16, 16)] = jnp.where(i == sel, NEG_INF, v)
            msk = lane < K_SEL
            acc = jnp.zeros((16,), jnp.float32)
            for c in range(n_cls):
                g = plsc.load_gather(p_v, [selv * n_cls + c])
                s = _all_reduce(jnp.where(msk, g, 0.0), jnp.add, lane)
                acc = jnp.where(lane == c, s / jnp.float32(K_SEL), acc)
            pltpu.sync_copy(b2_hbm, st_v)
            st_v[...] = st_v[...] + acc
            pltpu.sync_copy(st_v, out_hbm)

    return sc_kernel(score, p_flat, b2p)


def kernel(instances, bag_label, W1, b1, W2, b2):
    n = instances.shape[0]
    h, p = _features(instances, W1, b1.reshape(1, -1), W2)
    sign = jnp.where(jnp.asarray(bag_label) == 1, 1.0, -1.0)
    score = _typicality(sign.astype(jnp.float32).reshape(1), h).reshape(n)
    out16 = _sc_select(score, p.reshape(-1), jnp.pad(b2, (0, 16 - b2.shape[0])),
                       p.shape[1])
    return out16[: b2.shape[0]]


# speedup vs baseline: 1.4488x; 1.4410x over previous
import functools

import jax
import jax.numpy as jnp
from jax import lax
from jax.experimental import pallas as pl
from jax.experimental.pallas import tpu as pltpu
from jax.experimental.pallas import tpu_sc as plsc

EPSILON = 5.0
K_SEL = 10
NEG_INF = jnp.float32(-3.0e38)
BIG_I32 = jnp.int32(1 << 30)

_BM = 512


def _ka_body(x_ref, w1_ref, b1_ref, w2_ref, h_ref, p_ref):
    x = x_ref[...]
    h = jax.lax.dot_general(
        x, w1_ref[...], (((1,), (0,)), ((), ())),
        preferred_element_type=jnp.float32)
    h = jnp.maximum(h + b1_ref[...], 0.0)
    h_ref[...] = h
    p_ref[...] = jax.lax.dot_general(
        h, w2_ref[...], (((1,), (0,)), ((), ())),
        preferred_element_type=jnp.float32)


def _kbc_body(sign_ref, h_ref, score_ref, cnt_v, sc_v, sums_ref, band_ref,
              flag_ref):
    ph = pl.program_id(0)
    i = pl.program_id(1)
    j = pl.program_id(2)
    nj = pl.num_programs(2)
    n = h_ref.shape[0]

    def dists():
        hi = h_ref[pl.ds(i * _BM, _BM), :]
        hj = h_ref[pl.ds(j * _BM, _BM), :]
        aa = jnp.sum(hi * hi, axis=1, keepdims=True)
        bb = jnp.sum(hj * hj, axis=1)
        g = jax.lax.dot_general(
            hi, hj, (((1,), (1,)), ((), ())),
            preferred_element_type=jnp.float32)
        d2 = (aa + bb[None, :]) - 2.0 * g
        return jnp.sqrt(jnp.maximum(d2, 0.0))

    @pl.when(ph == 0)
    def _():
        d = dists()
        mask = d < EPSILON
        s1 = jnp.sum(d)
        s2 = jnp.sum(d * d)
        rc = jnp.sum(mask.astype(jnp.float32), axis=1, keepdims=True)
        flag_ref[i * nj + j] = jnp.any(mask).astype(jnp.int32)

        @pl.when((i == 0) & (j == 0))
        def _():
            sums_ref[0] = s1
            sums_ref[1] = s2

        @pl.when((i != 0) | (j != 0))
        def _():
            sums_ref[0] += s1
            sums_ref[1] += s2

        @pl.when(j == 0)
        def _():
            cnt_v[pl.ds(i * _BM, _BM), :] = rc

        @pl.when(j != 0)
        def _():
            cnt_v[pl.ds(i * _BM, _BM), :] += rc

    @pl.when(ph == 1)
    def _():
        @pl.when((i == 0) & (j == 0))
        def _():
            m = jnp.float32(n) * jnp.float32(n)
            mean = sums_ref[0] / m
            var = sums_ref[1] / m - mean * mean
            s = jnp.sqrt(jnp.maximum(var, 0.0))
            s = jnp.where(s < 1e-6, mean + 1e-6, s)
            band = jnp.maximum(1.06 * s * (float(n) ** (-0.2)), 0.001)
            band_ref[0] = 1.0 / (2.0 * band * band)

        @pl.when(j == 0)
        def _():
            sc_v[...] = jnp.zeros_like(sc_v)

        @pl.when(flag_ref[i * nj + j] == 1)
        def _():
            d = dists()
            mask = d < EPSILON
            gk = jnp.exp(-(d * d) * band_ref[0])
            sc_v[...] += jnp.sum(jnp.where(mask, gk, 0.0), axis=1,
                                 keepdims=True)

        @pl.when(j == nj - 1)
        def _():
            cnt = cnt_v[pl.ds(i * _BM, _BM), :]
            typ = jnp.where(cnt > 0.0, sc_v[...] / jnp.maximum(cnt, 1.0), 0.0)
            score_ref[...] = sign_ref[0] * typ


def _features(instances, W1, b1r, W2):
    n, d_in = instances.shape
    d_h = W1.shape[1]
    n_cls = W2.shape[1]
    nb = n // _BM
    return pl.pallas_call(
        _ka_body,
        grid=(nb,),
        in_specs=[
            pl.BlockSpec((_BM, d_in), lambda i: (i, 0)),
            pl.BlockSpec((d_in, d_h), lambda i: (0, 0)),
            pl.BlockSpec((1, d_h), lambda i: (0, 0)),
            pl.BlockSpec((d_h, n_cls), lambda i: (0, 0)),
        ],
        out_specs=[
            pl.BlockSpec((_BM, d_h), lambda i: (i, 0)),
            pl.BlockSpec((_BM, n_cls), lambda i: (i, 0)),
        ],
        out_shape=[
            jax.ShapeDtypeStruct((n, d_h), jnp.float32),
            jax.ShapeDtypeStruct((n, n_cls), jnp.float32),
        ],
    )(instances, W1, b1r, W2)


def _typicality(sign, h):
    n, d_h = h.shape
    nb = n // _BM
    return pl.pallas_call(
        _kbc_body,
        grid=(2, nb, nb),
        in_specs=[
            pl.BlockSpec(memory_space=pltpu.SMEM),
            pl.BlockSpec((n, d_h), lambda p, i, j: (0, 0)),
        ],
        out_specs=pl.BlockSpec((_BM, 1), lambda p, i, j: (i, 0)),
        out_shape=jax.ShapeDtypeStruct((n, 1), jnp.float32),
        scratch_shapes=[
            pltpu.VMEM((n, 1), jnp.float32),
            pltpu.VMEM((_BM, 1), jnp.float32),
            pltpu.SMEM((2,), jnp.float32),
            pltpu.SMEM((1,), jnp.float32),
            pltpu.SMEM((nb * nb,), jnp.int32),
        ],
    )(sign, h)


_NC, _NS = 1, 16
_NW = _NC * _NS


def _shuffle(x, idx):
    return x.at[idx].get(mode="promise_in_bounds")


def _all_reduce(v, op, lane):
    for s in (1, 2, 4, 8):
        v = op(v, _shuffle(v, lane ^ s))
    return v


def _sc_select(score, p_flat, b2p, n_cls):
    n = score.shape[0]
    chunk = n // _NW
    nregs = chunk // 16
    mesh = plsc.VectorSubcoreMesh(core_axis_name="c", subcore_axis_name="s",
                                  num_cores=_NC)

    @functools.partial(
        pl.kernel,
        mesh=mesh,
        compiler_params=pltpu.CompilerParams(needs_layout_passes=False),
        out_type=jax.ShapeDtypeStruct((16,), jnp.float32),
        scratch_types=[
            pltpu.VMEM((chunk,), jnp.float32),
            pltpu.VMEM((16,), jnp.float32),
            pltpu.VMEM((16,), jnp.int32),
            pltpu.VMEM_SHARED((_NW * 16,), jnp.float32),
            pltpu.VMEM_SHARED((_NW * 16,), jnp.int32),
            pltpu.VMEM((_NW * 16,), jnp.float32),
            pltpu.VMEM((_NW * 16,), jnp.int32),
            pltpu.VMEM((n * n_cls,), jnp.float32),
            pltpu.VMEM((16,), jnp.float32),
        ],
    )
    def sc_kernel(score_hbm, p_hbm, b2_hbm, out_hbm,
                  score_v, cv_v, ci_v, shv, shi, av, ai, p_v, st_v):
        wid = lax.axis_index("s")
        lane = lax.iota(jnp.int32, 16)
        pltpu.sync_copy(score_hbm.at[pl.ds(wid * chunk, chunk)], score_v)

        cv = jnp.full((16,), NEG_INF, jnp.float32)
        ci = jnp.full((16,), BIG_I32, jnp.int32)
        for t in range(K_SEL):
            best_v = jnp.full((16,), NEG_INF, jnp.float32)
            best_i = jnp.full((16,), BIG_I32, jnp.int32)
            for r in range(nregs):
                v = score_v[pl.ds(r * 16, 16)]
                idx = lane + (r * 16)
                upd = (v > best_v) | ((v == best_v) & (idx < best_i))
                best_v = jnp.where(upd, v, best_v)
                best_i = jnp.where(upd, idx, best_i)
            m = _all_reduce(best_v, jnp.maximum, lane)
            sel = _all_reduce(
                jnp.where(best_v == m, best_i, BIG_I32), jnp.minimum, lane)
            cv = jnp.where(lane == t, m, cv)
            ci = jnp.where(lane == t, sel + wid * chunk, ci)
            plsc.store_scatter(
                score_v, [sel],
                jnp.full((16,), NEG_INF, jnp.float32), mask=lane == 0)
        cv_v[...] = cv
        ci_v[...] = ci
        pltpu.sync_copy(cv_v, shv.at[pl.ds(wid * 16, 16)])
        pltpu.sync_copy(ci_v, shi.at[pl.ds(wid * 16, 16)])
        plsc.subcore_barrier()

        @pl.when(wid == 0)
        def _():
            pltpu.sync_copy(shv, av)
            pltpu.sync_copy(shi, ai)
            pltpu.sync_copy(p_hbm, p_v)
            selv = jnp.zeros((16,), jnp.int32)
            for t in range(K_SEL):
                best_v = jnp.full((16,), NEG_INF, jnp.float32)
                best_i = jnp.full((16,), BIG_I32, jnp.int32)
                for r in range(_NW):
                    v = av[pl.ds(r * 16, 16)]
                    i = ai[pl.ds(r * 16, 16)]
                    upd = (v > best_v) | ((v == best_v) & (i < best_i))
                    best_v = jnp.where(upd, v, best_v)
                    best_i = jnp.where(upd, i, best_i)
                m = _all_reduce(best_v, jnp.maximum, lane)
                sel = _all_reduce(
                    jnp.where(best_v == m, best_i, BIG_I32), jnp.minimum, lane)
                selv = jnp.where(lane == t, sel, selv)
                for r in range(_NW):
                    i = ai[pl.ds(r * 16, 16)]
                    v = av[pl.ds(r * 16, 16)]
                    av[pl.ds(r * 16, 16)] = jnp.where(i == sel, NEG_INF, v)
            msk = lane < K_SEL
            acc = jnp.zeros((16,), jnp.float32)
            for c in range(n_cls):
                g = plsc.load_gather(p_v, [selv * n_cls + c])
                s = _all_reduce(jnp.where(msk, g, 0.0), jnp.add, lane)
                acc = jnp.where(lane == c, s / jnp.float32(K_SEL), acc)
            pltpu.sync_copy(b2_hbm, st_v)
            st_v[...] = st_v[...] + acc
            pltpu.sync_copy(st_v, out_hbm)

    return sc_kernel(score, p_flat, b2p)


def kernel(instances, bag_label, W1, b1, W2, b2):
    n = instances.shape[0]
    h, p = _features(instances, W1, b1.reshape(1, -1), W2)
    sign = jnp.where(jnp.asarray(bag_label) == 1, 1.0, -1.0)
    score = _typicality(sign.astype(jnp.float32).reshape(1), h).reshape(n)
    out16 = _sc_select(score, p.reshape(-1), jnp.pad(b2, (0, 16 - b2.shape[0])),
                       p.shape[1])
    return out16[: b2.shape[0]]
